# baseline (speedup 1.0000x reference)
import functools

import jax
import jax.numpy as jnp
from jax import lax
from jax.experimental import pallas as pl
from jax.experimental.pallas import tpu as pltpu
from jax.experimental.pallas import tpu_sc as plsc

N = 10000
E = 320000
D = 128
NUM_LAYERS = 4
INV_AVG_DEG = float(N) / float(E)

NC = 2
NS = 16
NW = NC * NS
CHUNK = 64
EPW = 10240
NCHUNK = EPW // CHUNK
EPAD = NW * EPW
WIN = 8
NWIN = NCHUNK // WIN
NPAD = 10240
RPS = NPAD // NS
ZROWS = 64


_sc_mesh = plsc.VectorSubcoreMesh(core_axis_name="c", subcore_axis_name="s")


@functools.partial(
    pl.kernel,
    mesh=_sc_mesh,
    out_type=jax.ShapeDtypeStruct((NC, NPAD, D), jnp.float32),
    scratch_types=[
        pltpu.VMEM((3, WIN, CHUNK), jnp.int32),
        pltpu.VMEM((3, WIN, CHUNK), jnp.int32),
        pltpu.VMEM((CHUNK, D), jnp.float32),
        pltpu.VMEM((CHUNK, D), jnp.float32),
        pltpu.VMEM((CHUNK, D), jnp.float32),
        pltpu.VMEM((CHUNK, D), jnp.float32),
        pltpu.VMEM_SHARED((NPAD, D), jnp.float32),
        pltpu.SemaphoreType.DMA,
        pltpu.SemaphoreType.DMA,
        pltpu.SemaphoreType.DMA,
        pltpu.SemaphoreType.DMA,
        pltpu.SemaphoreType.DMA,
        pltpu.SemaphoreType.DMA,
        pltpu.SemaphoreType.DMA,
        pltpu.SemaphoreType.DMA,
        pltpu.SemaphoreType.DMA,
    ],
)
def _segsum_sc(table, src, dst, out, srcw, dstw, rows_a, rows_b, rows_c,
               rows_d, acc, sem_a, sem_b, sem_c, sem_d, ssem_a, ssem_b,
               ssem_c, ssem_d, sem_w):
    c = lax.axis_index("c")
    s = lax.axis_index("s")
    wid = c * NS + s

    for r in range(ZROWS):
        for q in range(D // 16):
            rows_a[r, pl.ds(q * 16, 16)] = jnp.zeros((16,), jnp.float32)
    base = s * RPS
    for b in range(RPS // ZROWS):
        pltpu.sync_copy(rows_a.at[pl.ds(0, ZROWS)],
                        acc.at[pl.ds(base + b * ZROWS, ZROWS)])
    plsc.subcore_barrier()

    def win_src(w):
        return src.at[wid, pl.ds(pl.multiple_of(w * jnp.int32(WIN), WIN), WIN)]

    def win_dst(w):
        return dst.at[wid, pl.ds(pl.multiple_of(w * jnp.int32(WIN), WIN), WIN)]

    def slot(w):
        return lax.rem(w, jnp.int32(3))

    pltpu.sync_copy(win_src(jnp.int32(0)), srcw.at[jnp.int32(0)])
    pltpu.sync_copy(win_dst(jnp.int32(0)), dstw.at[jnp.int32(0)])
    pltpu.async_copy(win_src(jnp.int32(1)), srcw.at[jnp.int32(1)], sem_w)
    pltpu.async_copy(win_dst(jnp.int32(1)), dstw.at[jnp.int32(1)], sem_w)

    def gather(j, rows, sem):
        w = lax.div(j, jnp.int32(WIN))
        pltpu.async_copy(
            table.at[srcw.at[slot(w), lax.rem(j, jnp.int32(WIN))]], rows, sem)

    def scat(j, rows):
        w = lax.div(j, jnp.int32(WIN))
        return (rows, acc.at[dstw.at[slot(w), lax.rem(j, jnp.int32(WIN))]])

    gather(jnp.int32(0), rows_a, sem_a)
    gather(jnp.int32(1), rows_b, sem_b)

    def chunk_body(j, carry):
        w = lax.div(j, jnp.int32(WIN))
        jw = lax.rem(j, jnp.int32(WIN))
        parity = lax.rem(j, jnp.int32(4))

        def step(rows, sem, ssem, rows_q, sem_q, ssem_q):
            pltpu.make_async_copy(
                table.at[srcw.at[slot(w), jw]], rows, sem).wait()
            sr, dr = scat(j, rows)
            pltpu.async_copy(sr, dr, ssem, add=True)

            @pl.when(j >= jnp.int32(2))
            def _():
                sr2, dr2 = scat(j - jnp.int32(2), rows_q)
                pltpu.make_async_copy(sr2, dr2, ssem_q).wait()

            @pl.when(j < jnp.int32(NCHUNK - 2))
            def _():
                gather(j + jnp.int32(2), rows_q, sem_q)

            @pl.when(jnp.logical_and(jw == jnp.int32(3),
                                     w < jnp.int32(NWIN - 1)))
            def _():
                wn = w + jnp.int32(1)
                pltpu.make_async_copy(win_src(wn), srcw.at[slot(wn)], sem_w).wait()
                pltpu.make_async_copy(win_dst(wn), dstw.at[slot(wn)], sem_w).wait()

            @pl.when(jnp.logical_and(jw == jnp.int32(4),
                                     w < jnp.int32(NWIN - 2)))
            def _():
                wn = w + jnp.int32(2)
                pltpu.async_copy(win_src(wn), srcw.at[slot(wn)], sem_w)
                pltpu.async_copy(win_dst(wn), dstw.at[slot(wn)], sem_w)

        @pl.when(parity == jnp.int32(0))
        def _():
            step(rows_a, sem_a, ssem_a, rows_c, sem_c, ssem_c)

        @pl.when(parity == jnp.int32(1))
        def _():
            step(rows_b, sem_b, ssem_b, rows_d, sem_d, ssem_d)

        @pl.when(parity == jnp.int32(2))
        def _():
            step(rows_c, sem_c, ssem_c, rows_a, sem_a, ssem_a)

        @pl.when(parity == jnp.int32(3))
        def _():
            step(rows_d, sem_d, ssem_d, rows_b, sem_b, ssem_b)

        return carry

    lax.fori_loop(jnp.int32(0), jnp.int32(NCHUNK), chunk_body, 0)

    sr, dr = scat(jnp.int32(NCHUNK - 2), rows_c)
    pltpu.make_async_copy(sr, dr, ssem_c).wait()
    sr, dr = scat(jnp.int32(NCHUNK - 1), rows_d)
    pltpu.make_async_copy(sr, dr, ssem_d).wait()
    plsc.subcore_barrier()

    pltpu.sync_copy(acc.at[pl.ds(base, RPS)], out.at[c, pl.ds(base, RPS)])


def _mm(a, b):
    return lax.dot_general(a, b, (((1,), (0,)), ((), ())),
                           preferred_element_type=jnp.float32)


def _extract(u, ew, eb, lw, lb):
    g = jnp.mean(u, axis=0, keepdims=True)
    o = _mm(g, ew) + eb
    return o + jnp.maximum(_mm(o, lw) + lb, 0.0)


def _pre_body(x_ref, iw_ref, ib_ref, u_ref, mux_ref):
    x = x_ref[...]
    u_ref[...] = _mm(x, iw_ref[...]) + ib_ref[...]
    mux_ref[...] = jnp.mean(x, axis=0, keepdims=True)


def _main_body(t_ref, p_ref, wself_ref, wmsg_ref, b_ref, gam_ref, bet_ref,
               t_out_ref, mu_ref):
    t = t_ref[...]
    agg = (p_ref[0, :N] + p_ref[1, :N]) * INV_AVG_DEG
    u = jnp.maximum(_mm(t, wself_ref[...]) + _mm(agg, wmsg_ref[...]) + b_ref[...], 0.0)
    mu = jnp.mean(u, axis=0, keepdims=True)
    mu_ref[...] = mu
    var = jnp.mean((u - mu) * (u - mu), axis=0, keepdims=True)
    t_out_ref[...] = (u - mu) / jnp.sqrt(var + 1e-5) * gam_ref[...] + bet_ref[...]


def _last_body(t_ref, p_ref, wself_ref, wmsg_ref, b_ref, mus_ref, ews_ref,
               ebs_ref, lws_ref, lbs_ref, aw_ref, ab_ref, fw_ref, fb_ref,
               out_ref):
    t = t_ref[...]
    agg = (p_ref[0, :N] + p_ref[1, :N]) * INV_AVG_DEG
    u = jnp.maximum(_mm(t, wself_ref[...]) + _mm(agg, wmsg_ref[...]) + b_ref[...], 0.0)
    mu4 = jnp.mean(u, axis=0, keepdims=True)

    def ext(k, g):
        o = _mm(g, ews_ref[k]) + ebs_ref[k, :1]
        return o + jnp.maximum(_mm(o, lws_ref[k]) + lbs_ref[k, :1], 0.0)

    o = ext(0, mus_ref[0:1, :])
    for k in range(1, NUM_LAYERS):
        o = o + ext(k, mus_ref[k:k + 1, :]) * (1.0 / NUM_LAYERS)
    o = o + ext(NUM_LAYERS, mu4) * (1.0 / NUM_LAYERS)

    o = jnp.maximum(_mm(o, aw_ref[...]) + ab_ref[...], 0.0) + o
    logits = _mm(o, fw_ref[...]) + fb_ref[...]
    m = jnp.max(logits, axis=-1, keepdims=True)
    sh = logits - m
    out_ref[...] = sh - jnp.log(jnp.sum(jnp.exp(sh), axis=-1, keepdims=True))


def _pre_call(x, iw, ib):
    return pl.pallas_call(
        _pre_body,
        out_shape=[
            jax.ShapeDtypeStruct((N, D), jnp.float32),
            jax.ShapeDtypeStruct((1, D), jnp.float32),
        ],
    )(x, iw, ib)


def _main_call(t, p, wself, wmsg, b, gam, bet):
    return pl.pallas_call(
        _main_body,
        out_shape=[
            jax.ShapeDtypeStruct((N, D), jnp.float32),
            jax.ShapeDtypeStruct((1, D), jnp.float32),
        ],
    )(t, p, wself, wmsg, b, gam, bet)


def _last_call(t, p, wself, wmsg, b, mus, ews, ebs, lws, lbs, aw, ab, fw, fb):
    return pl.pallas_call(
        _last_body,
        out_shape=jax.ShapeDtypeStruct((1, 10), jnp.float32),
    )(t, p, wself, wmsg, b, mus, ews, ebs, lws, lbs, aw, ab, fw, fb)


def kernel(x, params, edge_index):
    p = params
    pad_k = jnp.arange(EPAD - E, dtype=jnp.int32)
    src = jnp.concatenate(
        [edge_index[0].astype(jnp.int32), pad_k % jnp.int32(N)]
    ).reshape(NW, NCHUNK, CHUNK)
    dst = jnp.concatenate(
        [edge_index[1].astype(jnp.int32), jnp.int32(N) + pad_k % jnp.int32(NPAD - N)]
    ).reshape(NW, NCHUNK, CHUNK)

    def b2(v):
        return v.reshape(1, -1)

    t, mu_x = _pre_call(x, p['init_W'], b2(p['init_b']))
    mus = [mu_x]
    for i in range(NUM_LAYERS):
        lp = p['layers'][i]
        partials = _segsum_sc(t, src, dst)
        if i < NUM_LAYERS - 1:
            nxt = p['layers'][i + 1]
            t, mu = _main_call(t, partials, lp['Wself'], lp['Wmsg'], b2(lp['b']),
                               b2(nxt['bn_gamma']), b2(nxt['bn_beta']))
            mus.append(mu)
        else:
            ext_sets = [(p['noprop_ext_W'], p['noprop_ext_b'],
                         p['noprop_lin_W'], p['noprop_lin_b'])] + [
                (q['ext_W'], q['ext_b'], q['lin_W'], q['lin_b'])
                for q in p['layers']]
            ews = jnp.stack([e[0] for e in ext_sets])
            ebs = jnp.stack([b2(e[1]) for e in ext_sets])
            lws = jnp.stack([e[2] for e in ext_sets])
            lbs = jnp.stack([b2(e[3]) for e in ext_sets])
            out = _last_call(
                t, partials, lp['Wself'], lp['Wmsg'], b2(lp['b']),
                jnp.concatenate(mus, axis=0), ews, ebs, lws, lbs,
                p['after_W'], b2(p['after_b']),
                p['final_W'], b2(p['final_b']))
    return out


# speedup vs baseline: 1.2493x; 1.2493x over previous
import functools

import jax
import jax.numpy as jnp
from jax import lax
from jax.experimental import pallas as pl
from jax.experimental.pallas import tpu as pltpu
from jax.experimental.pallas import tpu_sc as plsc

N = 10000
E = 320000
D = 128
NUM_LAYERS = 4
INV_AVG_DEG = float(N) / float(E)

NC = 2
NS = 16
NW = NC * NS
CHUNK = 80
EPW = 10240
NCHUNK = EPW // CHUNK
EPAD = NW * EPW
WIN = 8
NWIN = NCHUNK // WIN
NPAD = 10240
RPS = NPAD // NS
ZROWS = 64


_sc_mesh = plsc.VectorSubcoreMesh(core_axis_name="c", subcore_axis_name="s")


@functools.partial(
    pl.kernel,
    mesh=_sc_mesh,
    out_type=jax.ShapeDtypeStruct((NC, NPAD, D), jnp.float32),
    scratch_types=[
        pltpu.VMEM((3, WIN, CHUNK), jnp.int32),
        pltpu.VMEM((3, WIN, CHUNK), jnp.int32),
        pltpu.VMEM((CHUNK, D), jnp.float32),
        pltpu.VMEM((CHUNK, D), jnp.float32),
        pltpu.VMEM((CHUNK, D), jnp.float32),
        pltpu.VMEM((CHUNK, D), jnp.float32),
        pltpu.VMEM_SHARED((NPAD, D), jnp.float32),
        pltpu.SemaphoreType.DMA,
        pltpu.SemaphoreType.DMA,
        pltpu.SemaphoreType.DMA,
        pltpu.SemaphoreType.DMA,
        pltpu.SemaphoreType.DMA,
    ],
)
def _segsum_sc(table, src, dst, out, srcw, dstw, rows_a, rows_b, rows_c,
               rows_d, acc, sem_a, sem_b, sem_c, sem_d, sem_w):
    c = lax.axis_index("c")
    s = lax.axis_index("s")
    wid = c * NS + s

    for r in range(ZROWS):
        for q in range(D // 16):
            rows_a[r, pl.ds(q * 16, 16)] = jnp.zeros((16,), jnp.float32)
    base = s * RPS
    for b in range(RPS // ZROWS):
        pltpu.sync_copy(rows_a.at[pl.ds(0, ZROWS)],
                        acc.at[pl.ds(base + b * ZROWS, ZROWS)])
    plsc.subcore_barrier()

    def win_src(w):
        return src.at[wid, pl.ds(pl.multiple_of(w * jnp.int32(WIN), WIN), WIN)]

    def win_dst(w):
        return dst.at[wid, pl.ds(pl.multiple_of(w * jnp.int32(WIN), WIN), WIN)]

    def slot(w):
        return lax.rem(w, jnp.int32(3))

    pltpu.sync_copy(win_src(jnp.int32(0)), srcw.at[jnp.int32(0)])
    pltpu.sync_copy(win_dst(jnp.int32(0)), dstw.at[jnp.int32(0)])
    pltpu.async_copy(win_src(jnp.int32(1)), srcw.at[jnp.int32(1)], sem_w)
    pltpu.async_copy(win_dst(jnp.int32(1)), dstw.at[jnp.int32(1)], sem_w)

    def gather(j, rows, sem):
        w = lax.div(j, jnp.int32(WIN))
        pltpu.async_copy(
            table.at[srcw.at[slot(w), lax.rem(j, jnp.int32(WIN))]], rows, sem)

    gather(jnp.int32(0), rows_a, sem_a)
    gather(jnp.int32(1), rows_b, sem_b)
    gather(jnp.int32(2), rows_c, sem_c)
    gather(jnp.int32(3), rows_d, sem_d)

    def chunk_body(j, carry):
        w = lax.div(j, jnp.int32(WIN))
        jw = lax.rem(j, jnp.int32(WIN))
        parity = lax.rem(j, jnp.int32(4))

        def step(rows, sem):
            pltpu.make_async_copy(
                table.at[srcw.at[slot(w), jw]], rows, sem).wait()
            pltpu.sync_copy(rows, acc.at[dstw.at[slot(w), jw]], add=True)

            @pl.when(jnp.logical_and(jw == jnp.int32(3),
                                     w < jnp.int32(NWIN - 1)))
            def _():
                wn = w + jnp.int32(1)
                pltpu.make_async_copy(win_src(wn), srcw.at[slot(wn)], sem_w).wait()
                pltpu.make_async_copy(win_dst(wn), dstw.at[slot(wn)], sem_w).wait()

            @pl.when(jnp.logical_and(jw == jnp.int32(4),
                                     w < jnp.int32(NWIN - 2)))
            def _():
                wn = w + jnp.int32(2)
                pltpu.async_copy(win_src(wn), srcw.at[slot(wn)], sem_w)
                pltpu.async_copy(win_dst(wn), dstw.at[slot(wn)], sem_w)

            @pl.when(j < jnp.int32(NCHUNK - 4))
            def _():
                gather(j + jnp.int32(4), rows, sem)

        @pl.when(parity == jnp.int32(0))
        def _():
            step(rows_a, sem_a)

        @pl.when(parity == jnp.int32(1))
        def _():
            step(rows_b, sem_b)

        @pl.when(parity == jnp.int32(2))
        def _():
            step(rows_c, sem_c)

        @pl.when(parity == jnp.int32(3))
        def _():
            step(rows_d, sem_d)

        return carry

    lax.fori_loop(jnp.int32(0), jnp.int32(NCHUNK), chunk_body, 0)
    plsc.subcore_barrier()

    pltpu.sync_copy(acc.at[pl.ds(base, RPS)], out.at[c, pl.ds(base, RPS)])


def _mm(a, b):
    return lax.dot_general(a, b, (((1,), (0,)), ((), ())),
                           preferred_element_type=jnp.float32)


def _extract(u, ew, eb, lw, lb):
    g = jnp.mean(u, axis=0, keepdims=True)
    o = _mm(g, ew) + eb
    return o + jnp.maximum(_mm(o, lw) + lb, 0.0)


def _pre_body(x_ref, iw_ref, ib_ref, u_ref, mux_ref):
    x = x_ref[...]
    u_ref[...] = _mm(x, iw_ref[...]) + ib_ref[...]
    mux_ref[...] = jnp.mean(x, axis=0, keepdims=True)


def _main_body(t_ref, p_ref, wself_ref, wmsg_ref, b_ref, gam_ref, bet_ref,
               t_out_ref, mu_ref):
    t = t_ref[...]
    agg = (p_ref[0, :N] + p_ref[1, :N]) * INV_AVG_DEG
    u = jnp.maximum(_mm(t, wself_ref[...]) + _mm(agg, wmsg_ref[...]) + b_ref[...], 0.0)
    mu = jnp.mean(u, axis=0, keepdims=True)
    mu_ref[...] = mu
    var = jnp.mean((u - mu) * (u - mu), axis=0, keepdims=True)
    t_out_ref[...] = (u - mu) / jnp.sqrt(var + 1e-5) * gam_ref[...] + bet_ref[...]


def _last_body(t_ref, p_ref, wself_ref, wmsg_ref, b_ref, mus_ref, ews_ref,
               ebs_ref, lws_ref, lbs_ref, aw_ref, ab_ref, fw_ref, fb_ref,
               out_ref):
    t = t_ref[...]
    agg = (p_ref[0, :N] + p_ref[1, :N]) * INV_AVG_DEG
    u = jnp.maximum(_mm(t, wself_ref[...]) + _mm(agg, wmsg_ref[...]) + b_ref[...], 0.0)
    mu4 = jnp.mean(u, axis=0, keepdims=True)

    def ext(k, g):
        o = _mm(g, ews_ref[k]) + ebs_ref[k, :1]
        return o + jnp.maximum(_mm(o, lws_ref[k]) + lbs_ref[k, :1], 0.0)

    o = ext(0, mus_ref[0:1, :])
    for k in range(1, NUM_LAYERS):
        o = o + ext(k, mus_ref[k:k + 1, :]) * (1.0 / NUM_LAYERS)
    o = o + ext(NUM_LAYERS, mu4) * (1.0 / NUM_LAYERS)

    o = jnp.maximum(_mm(o, aw_ref[...]) + ab_ref[...], 0.0) + o
    logits = _mm(o, fw_ref[...]) + fb_ref[...]
    m = jnp.max(logits, axis=-1, keepdims=True)
    sh = logits - m
    out_ref[...] = sh - jnp.log(jnp.sum(jnp.exp(sh), axis=-1, keepdims=True))


def _pre_call(x, iw, ib):
    return pl.pallas_call(
        _pre_body,
        out_shape=[
            jax.ShapeDtypeStruct((N, D), jnp.float32),
            jax.ShapeDtypeStruct((1, D), jnp.float32),
        ],
    )(x, iw, ib)


def _main_call(t, p, wself, wmsg, b, gam, bet):
    return pl.pallas_call(
        _main_body,
        out_shape=[
            jax.ShapeDtypeStruct((N, D), jnp.float32),
            jax.ShapeDtypeStruct((1, D), jnp.float32),
        ],
    )(t, p, wself, wmsg, b, gam, bet)


def _last_call(t, p, wself, wmsg, b, mus, ews, ebs, lws, lbs, aw, ab, fw, fb):
    return pl.pallas_call(
        _last_body,
        out_shape=jax.ShapeDtypeStruct((1, 10), jnp.float32),
    )(t, p, wself, wmsg, b, mus, ews, ebs, lws, lbs, aw, ab, fw, fb)


def kernel(x, params, edge_index):
    p = params
    pad_k = jnp.arange(EPAD - E, dtype=jnp.int32)
    src = jnp.concatenate(
        [edge_index[0].astype(jnp.int32), pad_k % jnp.int32(N)]
    ).reshape(NW, NCHUNK, CHUNK)
    dst = jnp.concatenate(
        [edge_index[1].astype(jnp.int32), jnp.int32(N) + pad_k % jnp.int32(NPAD - N)]
    ).reshape(NW, NCHUNK, CHUNK)

    def b2(v):
        return v.reshape(1, -1)

    t, mu_x = _pre_call(x, p['init_W'], b2(p['init_b']))
    mus = [mu_x]
    for i in range(NUM_LAYERS):
        lp = p['layers'][i]
        partials = _segsum_sc(t, src, dst)
        if i < NUM_LAYERS - 1:
            nxt = p['layers'][i + 1]
            t, mu = _main_call(t, partials, lp['Wself'], lp['Wmsg'], b2(lp['b']),
                               b2(nxt['bn_gamma']), b2(nxt['bn_beta']))
            mus.append(mu)
        else:
            ext_sets = [(p['noprop_ext_W'], p['noprop_ext_b'],
                         p['noprop_lin_W'], p['noprop_lin_b'])] + [
                (q['ext_W'], q['ext_b'], q['lin_W'], q['lin_b'])
                for q in p['layers']]
            ews = jnp.stack([e[0] for e in ext_sets])
            ebs = jnp.stack([b2(e[1]) for e in ext_sets])
            lws = jnp.stack([e[2] for e in ext_sets])
            lbs = jnp.stack([b2(e[3]) for e in ext_sets])
            out = _last_call(
                t, partials, lp['Wself'], lp['Wmsg'], b2(lp['b']),
                jnp.concatenate(mus, axis=0), ews, ebs, lws, lbs,
                p['after_W'], b2(p['after_b']),
                p['final_W'], b2(p['final_b']))
    return out
